# baseline (speedup 1.0000x reference)
import numpy as np
import jax
import jax.numpy as jnp
from jax import lax
from jax.experimental import pallas as pl
from jax.experimental.pallas import tpu as pltpu
from jax.experimental.pallas import tpu_sc as plsc

N = 10000
E = 160000
H = 256
HH = 128
OUT = 128
L = 3

NB = 5
BN = N // NB

NTILE = 16
CHUNK = 80
NCH = 128
EP = NTILE * NCH * CHUNK
ROWS_Z = 632
NACC = 10016


def _leaky(x):
    return jnp.where(x >= 0, x, 0.2 * x)


def _prep_body(nf_ref, dA_ref, bA_ref, h_ref, cs_ref):
    b = pl.program_id(0)
    y = jnp.dot(nf_ref[...], dA_ref[...], preferred_element_type=jnp.float32)
    y = y + bA_ref[...]
    h_ref[...] = y
    s = jnp.sum(y, axis=0, keepdims=True)

    @pl.when(b == 0)
    def _():
        cs_ref[...] = s

    @pl.when(b > 0)
    def _():
        cs_ref[...] = cs_ref[...] + s


_PREP_KW = dict(
    grid=(NB,),
    in_specs=[
        pl.BlockSpec((BN, 16), lambda b: (b, 0)),
        pl.BlockSpec((16, H), lambda b: (0, 0)),
        pl.BlockSpec((1, H), lambda b: (0, 0)),
    ],
    out_specs=[
        pl.BlockSpec((BN, H), lambda b: (b, 0)),
        pl.BlockSpec((1, H), lambda b: (0, 0)),
    ],
    out_shape=[
        jax.ShapeDtypeStruct((N, H), jnp.float32),
        jax.ShapeDtypeStruct((1, H), jnp.float32),
    ],
)
_prep = pl.pallas_call(_prep_body, **_PREP_KW)


def _expand_body(h_ref, e8_ref, r2_ref):
    r2_ref[...] = jnp.maximum(h_ref[...] + e8_ref[0], 0.0)


_EXPAND_KW = dict(
    grid=(NB, 2, 8),
    in_specs=[
        pl.BlockSpec((BN, HH), lambda b, hf, c: (b, hf)),
        pl.BlockSpec((1, 1, HH), lambda b, hf, c: (c * 2 + hf, 0, 0)),
    ],
    out_specs=pl.BlockSpec((BN, HH),
                           lambda b, hf, c: (hf * 8 * NB + c * NB + b, 0)),
    out_shape=jax.ShapeDtypeStruct((16 * N, HH), jnp.float32),
)
_expand = pl.pallas_call(_expand_body, **_EXPAND_KW)


def _mlp_body(h_ref, al_ref, ar_ref, w1_ref, b1_ref, w2_ref, b2_ref,
              hn_ref, cs_ref):
    b = pl.program_id(0)
    x = h_ref[...] + jnp.concatenate([al_ref[0], ar_ref[0]], axis=1)
    t = _leaky(jnp.dot(x, w1_ref[...], preferred_element_type=jnp.float32)
               + b1_ref[...])
    y = _leaky(jnp.dot(t, w2_ref[...], preferred_element_type=jnp.float32)
               + b2_ref[...])
    hn_ref[...] = y
    s = jnp.sum(y, axis=0, keepdims=True)

    @pl.when(b == 0)
    def _():
        cs_ref[...] = s

    @pl.when(b > 0)
    def _():
        cs_ref[...] = cs_ref[...] + s


_MLP_KW = dict(
    grid=(NB,),
    in_specs=[
        pl.BlockSpec((BN, H), lambda b: (b, 0)),
        pl.BlockSpec((1, BN, HH), lambda b: (0, b, 0)),
        pl.BlockSpec((1, BN, HH), lambda b: (1, b, 0)),
        pl.BlockSpec((H, H), lambda b: (0, 0)),
        pl.BlockSpec((1, H), lambda b: (0, 0)),
        pl.BlockSpec((H, H), lambda b: (0, 0)),
        pl.BlockSpec((1, H), lambda b: (0, 0)),
    ],
    out_specs=[
        pl.BlockSpec((BN, H), lambda b: (b, 0)),
        pl.BlockSpec((1, H), lambda b: (0, 0)),
    ],
    out_shape=[
        jax.ShapeDtypeStruct((N, H), jnp.float32),
        jax.ShapeDtypeStruct((1, H), jnp.float32),
    ],
)
_mlp = pl.pallas_call(_mlp_body, **_MLP_KW)


def _final_body(cs_ref, wr_ref, br_ref, o_ref):
    o_ref[...] = (jnp.dot(cs_ref[...], wr_ref[...],
                          preferred_element_type=jnp.float32)
                  + float(N) * jnp.sum(br_ref[...], axis=0, keepdims=True))


_FINAL_KW = dict(
    out_shape=jax.ShapeDtypeStruct((1, OUT), jnp.float32),
)
_final = pl.pallas_call(_final_body, **_FINAL_KW)


NBUF = 4
NSLOT = 8
ROWS_LAST = NACC - 15 * ROWS_Z


def _sc_edge_body(r2_hbm, idx_hbm, zeros_hbm, out_hbm,
                  idx_v, rows_v, acc_sh, isem, gsem, ssem):
    c = lax.axis_index("c")
    s = lax.axis_index("s")

    @pl.when(s < NTILE - 1)
    def _():
        pltpu.sync_copy(zeros_hbm.at[pl.ds(s * ROWS_Z, ROWS_Z)],
                        acc_sh.at[pl.ds(s * ROWS_Z, ROWS_Z)])

    @pl.when(s == NTILE - 1)
    def _():
        pltpu.sync_copy(zeros_hbm.at[pl.ds(15 * ROWS_Z, ROWS_LAST)],
                        acc_sh.at[pl.ds(15 * ROWS_Z, ROWS_LAST)])

    plsc.subcore_barrier()

    def load_idx(q, t):
        pltpu.async_copy(idx_hbm.at[c, s, q], idx_v.at[t, pl.ds(0, 2)],
                         isem.at[t])

    def wait_idx(t):
        pltpu.make_async_copy(idx_hbm.at[0, s, 0], idx_v.at[t, pl.ds(0, 2)],
                              isem.at[t]).wait()

    def edge_loop(r_hbm):
        def start_gather(t, b):
            pltpu.async_copy(r_hbm.at[idx_v.at[t, 0]], rows_v.at[b],
                             gsem.at[b])

        def wait_gather(b):
            pltpu.make_async_copy(r_hbm.at[pl.ds(0, CHUNK)], rows_v.at[b],
                                  gsem.at[b]).wait()

        def start_scatter(t, b):
            pltpu.async_copy(rows_v.at[b], acc_sh.at[idx_v.at[t, 1]],
                             ssem.at[b], add=True)

        def wait_scatter(b):
            pltpu.make_async_copy(rows_v.at[b], acc_sh.at[pl.ds(0, CHUNK)],
                                  ssem.at[b]).wait()

        for t in range(NSLOT):
            load_idx(t, t)
        for b in range(2):
            wait_idx(b)
            start_gather(b, b)

        @pl.loop(0, NCH, step=NSLOT)
        def _(j):
            for bb in range(NSLOT):
                q = j + bb
                b = bb % NBUF
                wait_gather(b)
                start_scatter(bb, b)

                @pl.when(q >= 1)
                def _():
                    wait_scatter((bb + NBUF - 1) % NBUF)

                @pl.when(q + 7 < NCH)
                def _():
                    load_idx(q + 7, (bb + 7) % NSLOT)

                @pl.when(q + 2 < NCH)
                def _():
                    wait_idx((bb + 2) % NSLOT)
                    start_gather((bb + 2) % NSLOT, (bb + 2) % NBUF)

        wait_scatter((NCH - 1) % NBUF)

    edge_loop(r2_hbm)

    plsc.subcore_barrier()

    @pl.when(s < NTILE - 1)
    def _():
        pltpu.sync_copy(acc_sh.at[pl.ds(s * ROWS_Z, ROWS_Z)],
                        out_hbm.at[c, pl.ds(s * ROWS_Z, ROWS_Z)])

    @pl.when(s == NTILE - 1)
    def _():
        pltpu.sync_copy(acc_sh.at[pl.ds(15 * ROWS_Z, ROWS_LAST)],
                        out_hbm.at[c, pl.ds(15 * ROWS_Z, ROWS_LAST)])


_SC_EDGE_CACHE = []


def _get_sc_edge():
    if not _SC_EDGE_CACHE:
        _SC_EDGE_CACHE.append(pl.kernel(
            _sc_edge_body,
            out_type=jax.ShapeDtypeStruct((2, NACC, HH), jnp.float32),
            mesh=plsc.VectorSubcoreMesh(core_axis_name="c",
                                        subcore_axis_name="s"),
            scratch_types=[
                pltpu.VMEM((NSLOT, 2, CHUNK), jnp.int32),
                pltpu.VMEM((NBUF, CHUNK, HH), jnp.float32),
                pltpu.VMEM_SHARED((NACC, HH), jnp.float32),
                pltpu.SemaphoreType.DMA((NSLOT,)),
                pltpu.SemaphoreType.DMA((NBUF,)),
                pltpu.SemaphoreType.DMA((NBUF,)),
            ],
        ))
    return _SC_EDGE_CACHE[0]


def kernel(nfeats, efeats, edge_index, atom_tables, bond_tables,
           Wc1, bc1, Wc2, bc2, Wr, br):
    f32 = jnp.float32

    nfp = jnp.pad(nfeats.astype(f32), ((0, 0), (0, 7)))
    dAp = jnp.pad(atom_tables[:, 1, :] - atom_tables[:, 0, :],
                  ((0, 7), (0, 0)))
    bA = jnp.sum(atom_tables[:, 0, :], axis=0, keepdims=True)

    combos = jnp.asarray(
        np.array([[(k >> j) & 1 for j in range(3)] for k in range(8)],
                 dtype=np.float32))
    dB = bond_tables[:, :, 1, :] - bond_tables[:, :, 0, :]
    bB = jnp.sum(bond_tables[:, :, 0, :], axis=1)
    E8 = bB[:, None, :] + jnp.einsum("kc,lch->lkh", combos, dB)

    src = edge_index[0].astype(jnp.int32)
    dst = edge_index[1].astype(jnp.int32)
    ef = efeats.astype(jnp.int32)
    code = ef[:, 0] + 2 * ef[:, 1] + 4 * ef[:, 2]
    gidx = jnp.pad(code * N + src, (0, EP - E)).reshape(NTILE, NCH, CHUNK)
    dstp = jnp.pad(dst, (0, EP - E),
                   constant_values=N).reshape(NTILE, NCH, CHUNK)
    idxs = jnp.stack([jnp.stack([gidx, dstp], axis=2),
                      jnp.stack([gidx + 8 * N, dstp], axis=2)], axis=0)
    zeros = jnp.zeros((NACC, HH), f32)

    sc_edge = _get_sc_edge()
    h, cs0 = _prep(nfp, dAp, bA)
    csums = [cs0]
    for i in range(L):
        e8i = E8[i].reshape(16, 1, HH)
        r2 = _expand(h, e8i)
        agg = sc_edge(r2, idxs, zeros)
        h, csi = _mlp(h, agg, agg, Wc1[i], bc1[i][None], Wc2[i],
                      bc2[i][None])
        csums.append(csi)

    csflat = jnp.concatenate(csums, axis=1)
    wrf = Wr.reshape((L + 1) * H, OUT)
    return _final(csflat, wrf, br)


# speedup vs baseline: 1.1308x; 1.1308x over previous
import numpy as np
import jax
import jax.numpy as jnp
from jax import lax
from jax.experimental import pallas as pl
from jax.experimental.pallas import tpu as pltpu
from jax.experimental.pallas import tpu_sc as plsc

N = 10000
E = 160000
H = 256
HH = 128
OUT = 128
L = 3

NB = 5
BN = N // NB

NTILE = 16
CLANE = 80
KROW = 1
CHUNK = KROW * CLANE
NCH = 128
EP = NTILE * NCH * CHUNK
ROWS_Z = 632
NACC = 10016
ROWS_LAST = NACC - 15 * ROWS_Z
NBUF = 4
NSLOT = 8


def _leaky(x):
    return jnp.where(x >= 0, x, 0.2 * x)


def _prep_body(nf_ref, dA_ref, bA_ref, h_ref, cs_ref):
    b = pl.program_id(0)
    y = jnp.dot(nf_ref[...], dA_ref[...], preferred_element_type=jnp.float32)
    y = y + bA_ref[...]
    h_ref[...] = y
    s = jnp.sum(y, axis=0, keepdims=True)

    @pl.when(b == 0)
    def _():
        cs_ref[...] = s

    @pl.when(b > 0)
    def _():
        cs_ref[...] = cs_ref[...] + s


_PREP_KW = dict(
    grid=(NB,),
    in_specs=[
        pl.BlockSpec((BN, 16), lambda b: (b, 0)),
        pl.BlockSpec((16, H), lambda b: (0, 0)),
        pl.BlockSpec((1, H), lambda b: (0, 0)),
    ],
    out_specs=[
        pl.BlockSpec((BN, H), lambda b: (b, 0)),
        pl.BlockSpec((1, H), lambda b: (0, 0)),
    ],
    out_shape=[
        jax.ShapeDtypeStruct((N, H), jnp.float32),
        jax.ShapeDtypeStruct((1, H), jnp.float32),
    ],
)
_prep = pl.pallas_call(_prep_body, **_PREP_KW)


def _expand_body(h_ref, e8_ref, rl_ref, rr_ref):
    r = jnp.maximum(h_ref[...] + e8_ref[0], 0.0)
    rl_ref[...] = r[:, :HH]
    rr_ref[...] = r[:, HH:]


_EXPAND_KW = dict(
    grid=(NB, 8),
    in_specs=[
        pl.BlockSpec((BN, H), lambda b, c: (b, 0)),
        pl.BlockSpec((1, 1, H), lambda b, c: (c, 0, 0)),
    ],
    out_specs=[
        pl.BlockSpec((BN, HH), lambda b, c: (c * NB + b, 0)),
        pl.BlockSpec((BN, HH), lambda b, c: (c * NB + b, 0)),
    ],
    out_shape=[
        jax.ShapeDtypeStruct((8 * N, HH), jnp.float32),
        jax.ShapeDtypeStruct((8 * N, HH), jnp.float32),
    ],
)
_expand = pl.pallas_call(_expand_body, **_EXPAND_KW)


def _mlp_body(h_ref, al_ref, ar_ref, w1_ref, b1_ref, w2_ref, b2_ref,
              hn_ref, cs_ref):
    b = pl.program_id(0)
    x = h_ref[...] + jnp.concatenate([al_ref[...], ar_ref[...]], axis=1)
    t = _leaky(jnp.dot(x, w1_ref[...], preferred_element_type=jnp.float32)
               + b1_ref[...])
    y = _leaky(jnp.dot(t, w2_ref[...], preferred_element_type=jnp.float32)
               + b2_ref[...])
    hn_ref[...] = y
    s = jnp.sum(y, axis=0, keepdims=True)

    @pl.when(b == 0)
    def _():
        cs_ref[...] = s

    @pl.when(b > 0)
    def _():
        cs_ref[...] = cs_ref[...] + s


_MLP_KW = dict(
    grid=(NB,),
    in_specs=[
        pl.BlockSpec((BN, H), lambda b: (b, 0)),
        pl.BlockSpec((BN, HH), lambda b: (b, 0)),
        pl.BlockSpec((BN, HH), lambda b: (b, 0)),
        pl.BlockSpec((H, H), lambda b: (0, 0)),
        pl.BlockSpec((1, H), lambda b: (0, 0)),
        pl.BlockSpec((H, H), lambda b: (0, 0)),
        pl.BlockSpec((1, H), lambda b: (0, 0)),
    ],
    out_specs=[
        pl.BlockSpec((BN, H), lambda b: (b, 0)),
        pl.BlockSpec((1, H), lambda b: (0, 0)),
    ],
    out_shape=[
        jax.ShapeDtypeStruct((N, H), jnp.float32),
        jax.ShapeDtypeStruct((1, H), jnp.float32),
    ],
)
_mlp = pl.pallas_call(_mlp_body, **_MLP_KW)


def _final_body(cs_ref, wr_ref, br_ref, o_ref):
    o_ref[...] = (jnp.dot(cs_ref[...], wr_ref[...],
                          preferred_element_type=jnp.float32)
                  + float(N) * jnp.sum(br_ref[...], axis=0, keepdims=True))


_FINAL_KW = dict(
    out_shape=jax.ShapeDtypeStruct((1, OUT), jnp.float32),
)
_final = pl.pallas_call(_final_body, **_FINAL_KW)


def _sc_edge_body(rl_hbm, rr_hbm, idx_hbm, zeros_hbm,
                  outl_hbm, outr_hbm,
                  idx_v, rows_v, acc_sh, isem, gsem, ssem):
    c = lax.axis_index("c")
    s = lax.axis_index("s")

    @pl.when(s < NTILE - 1)
    def _():
        pltpu.sync_copy(zeros_hbm.at[pl.ds(s * ROWS_Z, ROWS_Z)],
                        acc_sh.at[pl.ds(s * ROWS_Z, ROWS_Z)])

    @pl.when(s == NTILE - 1)
    def _():
        pltpu.sync_copy(zeros_hbm.at[pl.ds(15 * ROWS_Z, ROWS_LAST)],
                        acc_sh.at[pl.ds(15 * ROWS_Z, ROWS_LAST)])

    plsc.subcore_barrier()

    def load_idx(q, t):
        pltpu.async_copy(idx_hbm.at[s, q], idx_v.at[t], isem.at[t])

    def wait_idx(t):
        pltpu.make_async_copy(idx_hbm.at[s, 0], idx_v.at[t],
                              isem.at[t]).wait()

    def edge_loop(r_hbm):
        def start_gather(t, b):
            pltpu.async_copy(r_hbm.at[idx_v.at[t, 0]], rows_v.at[b],
                             gsem.at[b])

        def wait_gather(b):
            pltpu.make_async_copy(r_hbm.at[pl.ds(0, CHUNK)], rows_v.at[b],
                                  gsem.at[b]).wait()

        def start_scatter(t, b):
            pltpu.async_copy(rows_v.at[b], acc_sh.at[idx_v.at[t, 1]],
                             ssem.at[b], add=True)

        def wait_scatter(b):
            pltpu.make_async_copy(rows_v.at[b], acc_sh.at[pl.ds(0, CHUNK)],
                                  ssem.at[b]).wait()

        for t in range(NSLOT):
            load_idx(t, t)
        for b in range(2):
            wait_idx(b)
            start_gather(b, b)

        @pl.loop(0, NCH, step=NSLOT)
        def _(j):
            for bb in range(NSLOT):
                q = j + bb
                b = bb % NBUF
                wait_gather(b)
                start_scatter(bb, b)

                @pl.when(q >= 1)
                def _():
                    wait_scatter((bb + NBUF - 1) % NBUF)

                @pl.when(q + 7 < NCH)
                def _():
                    load_idx(q + 7, (bb + 7) % NSLOT)

                @pl.when(q + 2 < NCH)
                def _():
                    wait_idx((bb + 2) % NSLOT)
                    start_gather((bb + 2) % NSLOT, (bb + 2) % NBUF)

        wait_scatter((NCH - 1) % NBUF)

    @pl.when(c == 0)
    def _():
        edge_loop(rl_hbm)

    @pl.when(c == 1)
    def _():
        edge_loop(rr_hbm)

    plsc.subcore_barrier()

    def out_copy(out_hbm):
        @pl.when(s < NTILE - 1)
        def _():
            pltpu.sync_copy(acc_sh.at[pl.ds(s * ROWS_Z, ROWS_Z)],
                            out_hbm.at[pl.ds(s * ROWS_Z, ROWS_Z)])

        @pl.when(s == NTILE - 1)
        def _():
            pltpu.sync_copy(acc_sh.at[pl.ds(15 * ROWS_Z, ROWS_LAST)],
                            out_hbm.at[pl.ds(15 * ROWS_Z, ROWS_LAST)])

    @pl.when(c == 0)
    def _():
        out_copy(outl_hbm)

    @pl.when(c == 1)
    def _():
        out_copy(outr_hbm)


_SC_EDGE_CACHE = []


def _get_sc_edge():
    if not _SC_EDGE_CACHE:
        _SC_EDGE_CACHE.append(pl.kernel(
            _sc_edge_body,
            out_type=[
                jax.ShapeDtypeStruct((NACC, HH), jnp.float32),
                jax.ShapeDtypeStruct((NACC, HH), jnp.float32),
            ],
            mesh=plsc.VectorSubcoreMesh(core_axis_name="c",
                                        subcore_axis_name="s"),
            scratch_types=[
                pltpu.VMEM((NSLOT, 2, CLANE), jnp.int32),
                pltpu.VMEM((NBUF, CHUNK, HH), jnp.float32),
                pltpu.VMEM_SHARED((NACC, HH), jnp.float32),
                pltpu.SemaphoreType.DMA((NSLOT,)),
                pltpu.SemaphoreType.DMA((NBUF,)),
                pltpu.SemaphoreType.DMA((NBUF,)),
            ],
        ))
    return _SC_EDGE_CACHE[0]


def kernel(nfeats, efeats, edge_index, atom_tables, bond_tables,
           Wc1, bc1, Wc2, bc2, Wr, br):
    f32 = jnp.float32

    nfp = jnp.pad(nfeats.astype(f32), ((0, 0), (0, 7)))
    dAp = jnp.pad(atom_tables[:, 1, :] - atom_tables[:, 0, :],
                  ((0, 7), (0, 0)))
    bA = jnp.sum(atom_tables[:, 0, :], axis=0, keepdims=True)

    combos = jnp.asarray(
        np.array([[(k >> j) & 1 for j in range(3)] for k in range(8)],
                 dtype=np.float32))
    dB = bond_tables[:, :, 1, :] - bond_tables[:, :, 0, :]
    bB = jnp.sum(bond_tables[:, :, 0, :], axis=1)
    E8 = bB[:, None, :] + jnp.einsum("kc,lch->lkh", combos, dB)

    src = edge_index[0].astype(jnp.int32)
    dst = edge_index[1].astype(jnp.int32)
    ef = efeats.astype(jnp.int32)
    code = ef[:, 0] + 2 * ef[:, 1] + 4 * ef[:, 2]
    gidx = jnp.pad(code * N + src,
                   (0, EP - E)).reshape(NTILE, NCH, CLANE)
    dstp = jnp.pad(dst, (0, EP - E),
                   constant_values=N).reshape(NTILE, NCH, CLANE)
    idxs = jnp.stack([gidx, dstp], axis=2)
    zeros = jnp.zeros((NACC, HH), f32)

    sc_edge = _get_sc_edge()
    h, cs0 = _prep(nfp, dAp, bA)
    csums = [cs0]
    for i in range(L):
        rl, rr = _expand(h, E8[i][:, None, :])
        al, ar = sc_edge(rl, rr, idxs, zeros)
        h, csi = _mlp(h, al, ar, Wc1[i], bc1[i][None], Wc2[i], bc2[i][None])
        csums.append(csi)

    csflat = jnp.concatenate(csums, axis=1)
    wrf = Wr.reshape((L + 1) * H, OUT)
    return _final(csflat, wrf, br)
